# baseline (speedup 1.0000x reference)
import functools

import jax
import jax.numpy as jnp
from jax import lax
from jax.experimental import pallas as pl
from jax.experimental.pallas import tpu as pltpu
from jax.experimental.pallas import tpu_sc as plsc

N = 512
E = 32768
D = 128
G = 32

NC = 2
NS = 16
NW = NC * NS
EPT = E // NW
CHUNK = 128
NCHUNK = EPT // CHUNK
ROWS_PER_TILE = N // NS
WORDS_PER_TILE = ROWS_PER_TILE * N
ZW = 2048
CBLK = 4


def _stage_a(obs_ref, wemb_ref, bemb_ref, wgat_ref, asrc_ref, adst_ref,
             local_ref, h_ref, as_ref, ad_ref):
    dn = (((1,), (1,)), ((), ()))
    local = jnp.tanh(
        lax.dot_general(obs_ref[...], wemb_ref[...], dn,
                        preferred_element_type=jnp.float32) + bemb_ref[...])
    h = lax.dot_general(local, wgat_ref[...], dn,
                        preferred_element_type=jnp.float32)
    local_ref[...] = local
    h_ref[...] = h
    as_ref[...] = jnp.sum(h * asrc_ref[...][None, :], axis=1)
    ad_ref[...] = jnp.sum(h * adst_ref[...][None, :], axis=1)


def _sc_edge(edge_hbm, as_hbm, ad_hbm, out_hbm,
             srcv, dstv, asv, adv, wbuf, fbuf, zbuf, wacc,
             sem_z, sem_in, sem_sc, sem_out):
    cid = lax.axis_index("c")
    sid = lax.axis_index("s")
    wid = cid * NS + sid
    base = wid * EPT

    c_in = [
        pltpu.async_copy(edge_hbm.at[0, pl.ds(base, EPT)], srcv, sem_in),
        pltpu.async_copy(edge_hbm.at[1, pl.ds(base, EPT)], dstv, sem_in),
        pltpu.async_copy(as_hbm, asv, sem_in),
        pltpu.async_copy(ad_hbm, adv, sem_in),
    ]

    def zlane(t, _):
        zbuf[pl.ds(t * 16, 16)] = jnp.zeros((16,), jnp.float32)
        return _
    lax.fori_loop(0, ZW // 16, zlane, 0)
    c_z = [
        pltpu.async_copy(
            zbuf, wacc.at[pl.ds(sid * WORDS_PER_TILE + r * ZW, ZW)], sem_z)
        for r in range(WORDS_PER_TILE // ZW)
    ]
    for c in c_in:
        c.wait()

    for j in range(NCHUNK):
        def body(t, _):
            sl = pl.ds(j * CHUNK + t * 16, 16)
            sv = srcv[sl]
            dv = dstv[sl]
            av = plsc.load_gather(asv, [sv])
            bv = plsc.load_gather(adv, [dv])
            e = av + bv
            e = jnp.maximum(e, 0.2 * e)
            w = jnp.exp(e)
            off = t * 16
            wbuf[j, pl.ds(off, 16)] = w
            fbuf[j, pl.ds(off, 16)] = (dv << 9) | sv
            return _
        lax.fori_loop(0, CHUNK // 16, body, 0)

    for c in c_z:
        c.wait()
    plsc.subcore_barrier()

    c_sc = [
        pltpu.async_copy(wbuf.at[j], wacc.at[fbuf.at[j]], sem_sc, add=True)
        for j in range(NCHUNK)
    ]
    for c in c_sc:
        c.wait()
    plsc.subcore_barrier()

    pltpu.async_copy(
        wacc.at[pl.ds(sid * WORDS_PER_TILE, WORDS_PER_TILE)],
        out_hbm.at[wid],
        sem_out).wait()


def _stage_c(wp_ref, h_ref, local_ref, bgat_ref, core_ref, gid_ref,
             wq_ref, bq_ref, wk_ref, bk_ref, wv_ref, bv_ref, wo_ref, bo_ref,
             out_ref):
    dn = (((1,), (1,)), ((), ()))
    W = (wp_ref[0:NS, :].reshape(N, N)
         + wp_ref[NS:2 * NS, :].reshape(N, N))
    h = h_ref[...]
    num = jnp.dot(W, h, preferred_element_type=jnp.float32)
    den = jnp.sum(W, axis=1, keepdims=True)
    intra = jnp.tanh(num / (den + 1e-16) + bgat_ref[...])
    cn = core_ref[...]
    oh_c = (lax.broadcasted_iota(jnp.int32, (G, N), 1)
            == cn[:, None]).astype(jnp.float32)
    core = jnp.dot(oh_c, intra, preferred_element_type=jnp.float32)
    q = lax.dot_general(core, wq_ref[...], dn,
                        preferred_element_type=jnp.float32) + bq_ref[...]
    k = lax.dot_general(core, wk_ref[...], dn,
                        preferred_element_type=jnp.float32) + bk_ref[...]
    v = lax.dot_general(core, wv_ref[...], dn,
                        preferred_element_type=jnp.float32) + bv_ref[...]
    scores = lax.dot_general(q, k, dn, preferred_element_type=jnp.float32)
    scores = scores / jnp.sqrt(jnp.float32(D))
    scores = scores - jnp.max(scores, axis=1, keepdims=True)
    ex = jnp.exp(scores)
    attn = ex / jnp.sum(ex, axis=1, keepdims=True)
    av = jnp.dot(attn, v, preferred_element_type=jnp.float32)
    go = lax.dot_general(av, wo_ref[...], dn,
                         preferred_element_type=jnp.float32) + bo_ref[...]
    gid = gid_ref[...]
    oh_g = (lax.broadcasted_iota(jnp.int32, (N, G), 1)
            == gid[:, None]).astype(jnp.float32)
    inter = jnp.dot(oh_g, go, preferred_element_type=jnp.float32)
    out_ref[:, 0:D] = local_ref[...]
    out_ref[:, D:2 * D] = inter
    out_ref[:, 2 * D:3 * D] = intra


@functools.cache
def _sc_edge_call():
    return functools.partial(
        pl.kernel,
        mesh=plsc.VectorSubcoreMesh(core_axis_name="c", subcore_axis_name="s"),
        out_type=jax.ShapeDtypeStruct((NW, WORDS_PER_TILE), jnp.float32),
        compiler_params=pltpu.CompilerParams(needs_layout_passes=False),
        scratch_types=[
            pltpu.VMEM((EPT,), jnp.int32),
            pltpu.VMEM((EPT,), jnp.int32),
            pltpu.VMEM((N,), jnp.float32),
            pltpu.VMEM((N,), jnp.float32),
            pltpu.VMEM((NCHUNK, CHUNK), jnp.float32),
            pltpu.VMEM((NCHUNK, CHUNK), jnp.int32),
            pltpu.VMEM((ZW,), jnp.float32),
            pltpu.VMEM_SHARED((N * N,), jnp.float32),
            pltpu.SemaphoreType.DMA,
            pltpu.SemaphoreType.DMA,
            pltpu.SemaphoreType.DMA,
            pltpu.SemaphoreType.DMA,
        ],
    )(_sc_edge)


def kernel(local_obs, edge_index, core_node, group_ids, W_emb, b_emb, W_gat,
           att_src, att_dst, b_gat, Wq, bq, Wk, bk, Wv, bv, Wo, bo):
    f32 = jnp.float32
    local, h, a_s, a_d = pl.pallas_call(
        _stage_a,
        out_shape=[
            jax.ShapeDtypeStruct((N, D), f32),
            jax.ShapeDtypeStruct((N, D), f32),
            jax.ShapeDtypeStruct((N,), f32),
            jax.ShapeDtypeStruct((N,), f32),
        ],
    )(local_obs, W_emb, b_emb, W_gat, att_src, att_dst)

    w_partial = _sc_edge_call()(edge_index, a_s, a_d)

    out = pl.pallas_call(
        _stage_c,
        out_shape=jax.ShapeDtypeStruct((N, 3 * D), f32),
    )(w_partial, h, local, b_gat, core_node, group_ids,
      Wq, bq, Wk, bk, Wv, bv, Wo, bo)
    return out


# speedup vs baseline: 1.0803x; 1.0803x over previous
import functools

import jax
import jax.numpy as jnp
from jax import lax
from jax.experimental import pallas as pl
from jax.experimental.pallas import tpu as pltpu
from jax.experimental.pallas import tpu_sc as plsc

N = 512
E = 32768
D = 128
G = 32

NC = 2
NS = 16
NW = NC * NS
EPT = E // NW
CHUNK = 128
NCHUNK = EPT // CHUNK
ROWS_PER_TILE = N // NS
WORDS_PER_TILE = ROWS_PER_TILE * N
ZW = 2048
CBLK = 4


def _stage_a(obs_ref, wemb_ref, bemb_ref, wgat_ref, asrc_ref, adst_ref,
             local_ref, h_ref, as_ref, ad_ref):
    dn = (((1,), (1,)), ((), ()))
    local = jnp.tanh(
        lax.dot_general(obs_ref[...], wemb_ref[...], dn,
                        preferred_element_type=jnp.float32) + bemb_ref[...])
    h = lax.dot_general(local, wgat_ref[...], dn,
                        preferred_element_type=jnp.float32)
    local_ref[...] = local
    h_ref[...] = h
    as_ref[...] = jnp.sum(h * asrc_ref[...][None, :], axis=1)
    ad_ref[...] = jnp.sum(h * adst_ref[...][None, :], axis=1)


def _sc_edge(edge_hbm, out_hbm,
             srcv, dstv, wbuf, fbuf, zbuf, wacc,
             sem_z, sem_in, sem_sc, sem_out):
    cid = lax.axis_index("c")
    sid = lax.axis_index("s")
    wid = cid * NS + sid
    base = wid * EPT

    c_in = [
        pltpu.async_copy(edge_hbm.at[0, pl.ds(base, EPT)], srcv, sem_in),
        pltpu.async_copy(edge_hbm.at[1, pl.ds(base, EPT)], dstv, sem_in),
    ]

    def zlane(t, _):
        zbuf[pl.ds(t * 16, 16)] = jnp.zeros((16,), jnp.float32)
        return _
    lax.fori_loop(0, ZW // 16, zlane, 0)
    c_z = [
        pltpu.async_copy(
            zbuf, wacc.at[pl.ds(sid * WORDS_PER_TILE + r * ZW, ZW)], sem_z)
        for r in range(WORDS_PER_TILE // ZW)
    ]
    def olane(t, _):
        wbuf[0, pl.ds(t * 16, 16)] = jnp.ones((16,), jnp.float32)
        return _
    lax.fori_loop(0, CHUNK // 16, olane, 0)
    for c in c_in:
        c.wait()

    for j in range(NCHUNK):
        def body(t, _):
            sl = pl.ds(j * CHUNK + t * 16, 16)
            sv = srcv[sl]
            dv = dstv[sl]
            fbuf[j, pl.ds(t * 16, 16)] = (dv << 9) | sv
            return _
        lax.fori_loop(0, CHUNK // 16, body, 0)

    for c in c_z:
        c.wait()
    plsc.subcore_barrier()

    c_sc = [
        pltpu.async_copy(wbuf.at[0], wacc.at[fbuf.at[j]], sem_sc, add=True)
        for j in range(NCHUNK)
    ]
    for c in c_sc:
        c.wait()
    plsc.subcore_barrier()

    pltpu.async_copy(
        wacc.at[pl.ds(sid * WORDS_PER_TILE, WORDS_PER_TILE)],
        out_hbm.at[wid],
        sem_out).wait()


def _stage_c(wp_ref, as_ref, ad_ref, h_ref, local_ref, bgat_ref, core_ref, gid_ref,
             wq_ref, bq_ref, wk_ref, bk_ref, wv_ref, bv_ref, wo_ref, bo_ref,
             out_ref):
    dn = (((1,), (1,)), ((), ()))
    C = (wp_ref[0:NS, :].reshape(N, N)
         + wp_ref[NS:2 * NS, :].reshape(N, N))
    emat = ad_ref[...][:, None] + as_ref[...][None, :]
    emat = jnp.maximum(emat, 0.2 * emat)
    W = C * jnp.exp(emat)
    h = h_ref[...]
    num = jnp.dot(W, h, preferred_element_type=jnp.float32)
    den = jnp.sum(W, axis=1, keepdims=True)
    intra = jnp.tanh(num / (den + 1e-16) + bgat_ref[...])
    cn = core_ref[...]
    oh_c = (lax.broadcasted_iota(jnp.int32, (G, N), 1)
            == cn[:, None]).astype(jnp.float32)
    core = jnp.dot(oh_c, intra, preferred_element_type=jnp.float32)
    q = lax.dot_general(core, wq_ref[...], dn,
                        preferred_element_type=jnp.float32) + bq_ref[...]
    k = lax.dot_general(core, wk_ref[...], dn,
                        preferred_element_type=jnp.float32) + bk_ref[...]
    v = lax.dot_general(core, wv_ref[...], dn,
                        preferred_element_type=jnp.float32) + bv_ref[...]
    scores = lax.dot_general(q, k, dn, preferred_element_type=jnp.float32)
    scores = scores / jnp.sqrt(jnp.float32(D))
    scores = scores - jnp.max(scores, axis=1, keepdims=True)
    ex = jnp.exp(scores)
    attn = ex / jnp.sum(ex, axis=1, keepdims=True)
    av = jnp.dot(attn, v, preferred_element_type=jnp.float32)
    go = lax.dot_general(av, wo_ref[...], dn,
                         preferred_element_type=jnp.float32) + bo_ref[...]
    gid = gid_ref[...]
    oh_g = (lax.broadcasted_iota(jnp.int32, (N, G), 1)
            == gid[:, None]).astype(jnp.float32)
    inter = jnp.dot(oh_g, go, preferred_element_type=jnp.float32)
    out_ref[:, 0:D] = local_ref[...]
    out_ref[:, D:2 * D] = inter
    out_ref[:, 2 * D:3 * D] = intra


@functools.cache
def _sc_edge_call():
    return functools.partial(
        pl.kernel,
        mesh=plsc.VectorSubcoreMesh(core_axis_name="c", subcore_axis_name="s"),
        out_type=jax.ShapeDtypeStruct((NW, WORDS_PER_TILE), jnp.float32),
        compiler_params=pltpu.CompilerParams(needs_layout_passes=False),
        scratch_types=[
            pltpu.VMEM((EPT,), jnp.int32),
            pltpu.VMEM((EPT,), jnp.int32),
            pltpu.VMEM((1, CHUNK), jnp.float32),
            pltpu.VMEM((NCHUNK, CHUNK), jnp.int32),
            pltpu.VMEM((ZW,), jnp.float32),
            pltpu.VMEM_SHARED((N * N,), jnp.float32),
            pltpu.SemaphoreType.DMA,
            pltpu.SemaphoreType.DMA,
            pltpu.SemaphoreType.DMA,
            pltpu.SemaphoreType.DMA,
        ],
    )(_sc_edge)


def kernel(local_obs, edge_index, core_node, group_ids, W_emb, b_emb, W_gat,
           att_src, att_dst, b_gat, Wq, bq, Wk, bk, Wv, bv, Wo, bo):
    f32 = jnp.float32
    local, h, a_s, a_d = pl.pallas_call(
        _stage_a,
        out_shape=[
            jax.ShapeDtypeStruct((N, D), f32),
            jax.ShapeDtypeStruct((N, D), f32),
            jax.ShapeDtypeStruct((N,), f32),
            jax.ShapeDtypeStruct((N,), f32),
        ],
    )(local_obs, W_emb, b_emb, W_gat, att_src, att_dst)

    w_partial = _sc_edge_call()(edge_index)

    out = pl.pallas_call(
        _stage_c,
        out_shape=jax.ShapeDtypeStruct((N, 3 * D), f32),
    )(w_partial, a_s, a_d, h, local, b_gat, core_node, group_ids,
      Wq, bq, Wk, bk, Wv, bv, Wo, bo)
    return out
